# baseline (speedup 1.0000x reference)
import functools

import jax
import jax.numpy as jnp
from jax import lax
from jax.experimental import pallas as pl
from jax.experimental.pallas import tpu as pltpu
from jax.experimental.pallas import tpu_sc as plsc

_BLK = 128


@functools.lru_cache(maxsize=None)
def _build_gather(bsz: int, seq: int, d: int):
    info = plsc.get_sparse_core_info()
    nc, ns = info.num_cores, info.num_subcores
    nw = nc * ns
    assert bsz % (nw * _BLK) == 0
    b_per_w = bsz // nw
    nblk = b_per_w // _BLK
    nblocks = seq * nblk
    mesh = plsc.VectorSubcoreMesh(core_axis_name="c", subcore_axis_name="s")

    @functools.partial(
        pl.kernel,
        mesh=mesh,
        compiler_params=pltpu.CompilerParams(needs_layout_passes=False),
        out_type=jax.ShapeDtypeStruct((seq, d, bsz), jnp.float32),
        scratch_types=[
            pltpu.VMEM((seq * b_per_w,), jnp.int32),
            pltpu.VMEM((2, _BLK), jnp.int32),
            pltpu.VMEM((2, _BLK), jnp.int32),
            pltpu.VMEM((2, _BLK, 128), jnp.float32),
            pltpu.VMEM((2, d, _BLK), jnp.float32),
            pltpu.SemaphoreType.DMA((2,)),
            pltpu.SemaphoreType.DMA((2,)),
        ],
    )
    def k(ids_hbm, table_hbm, out_hbm, raw_v, idx_v, par_v, pairs_v, tile_v,
          gsem, wsem):
        wid = lax.axis_index("s") * nc + lax.axis_index("c")
        b0w = wid * b_per_w

        for s0 in range(seq):
            pltpu.async_copy(ids_hbm.at[s0, pl.ds(b0w, b_per_w)],
                             raw_v.at[pl.ds(s0 * b_per_w, b_per_w)],
                             gsem.at[0])
        for s0 in range(seq):
            pltpu.make_async_copy(ids_hbm.at[0, pl.ds(0, b_per_w)],
                                  raw_v.at[pl.ds(0, b_per_w)],
                                  gsem.at[0]).wait()
        bvecs = [jax.lax.iota(jnp.int32, 16) + (g * 16) for g in range(8)]

        def sb(blk):
            return blk // nblk, lax.rem(blk, nblk)

        def prep_and_gather(blk, sl):
            s, bb = sb(blk)
            base = pl.multiple_of(s * b_per_w, b_per_w) + pl.multiple_of(
                bb * _BLK, _BLK)
            for g in range(8):
                raw = raw_v[pl.ds(base + g * 16, 16)]
                idx_v.at[sl][pl.ds(g * 16, 16)] = lax.shift_right_logical(
                    raw, 1)
                par_v.at[sl][pl.ds(g * 16, 16)] = lax.shift_left(
                    lax.bitwise_and(raw, 1), 6)
            pltpu.async_copy(table_hbm.at[idx_v.at[sl]], pairs_v.at[sl],
                             gsem.at[sl])

        def wait_gather(sl):
            pltpu.make_async_copy(table_hbm.at[idx_v.at[sl]], pairs_v.at[sl],
                                  gsem.at[sl]).wait()

        def start_write(blk, sl):
            s, bb = sb(blk)
            pltpu.async_copy(tile_v.at[sl],
                             out_hbm.at[s, :, pl.ds(b0w + bb * _BLK, _BLK)],
                             wsem.at[sl])

        def wait_write(sl):
            pltpu.make_async_copy(tile_v.at[sl],
                                  out_hbm.at[0, :, pl.ds(0, _BLK)],
                                  wsem.at[sl]).wait()

        def transpose(sl):
            pairs = pairs_v.at[sl]
            tile = tile_v.at[sl]
            for g in range(8):
                par = par_v.at[sl][pl.ds(g * 16, 16)]
                bv = bvecs[g]

                @plsc.parallel_loop(0, d, unroll=16, carry=par)
                def col(c, pr):
                    v = plsc.load_gather(pairs, [bv, pr + c])
                    tile.at[c][pl.ds(g * 16, 16)] = v
                    return pr

        prep_and_gather(0, 0)

        def body(i, carry):
            for sl in range(2):
                blk = i * 2 + sl
                nxt = blk + 1

                @pl.when(nxt < nblocks)
                def _():
                    prep_and_gather(nxt, sl ^ 1)

                wait_gather(sl)

                @pl.when(blk >= 2)
                def _():
                    wait_write(sl)

                transpose(sl)
                start_write(blk, sl)
            return carry

        lax.fori_loop(0, nblocks // 2, body, 0)
        wait_write(0)
        wait_write(1)

    return k


def kernel(input_ids, table):
    b, s = input_ids.shape
    d = table.shape[1]
    ids_t = input_ids.T.astype(jnp.int32)
    tbl2 = table.reshape(table.shape[0] // 2, 2 * d)
    out = _build_gather(b, s, d)(ids_t, tbl2)
    return jnp.transpose(out, (2, 0, 1))


# speedup vs baseline: 1.0175x; 1.0175x over previous
import functools

import jax
import jax.numpy as jnp
from jax import lax
from jax.experimental import pallas as pl
from jax.experimental.pallas import tpu as pltpu
from jax.experimental.pallas import tpu_sc as plsc

_NBUF = 2
_CHUNK = 512
_ROW = 128


@functools.lru_cache(maxsize=None)
def _build_gather(n_total: int, d: int):
    info = plsc.get_sparse_core_info()
    nc, ns = info.num_cores, info.num_subcores
    nw = nc * ns
    assert n_total % nw == 0
    n_per_w = n_total // nw
    chunk = _CHUNK
    nbuf = _NBUF
    nstream = chunk // _ROW
    assert n_per_w % (chunk * nbuf) == 0
    n_chunks = n_per_w // chunk
    outer = n_chunks // nbuf
    idx_rows_w = n_per_w // _ROW
    mesh = plsc.VectorSubcoreMesh(core_axis_name="c", subcore_axis_name="s")

    @functools.partial(
        pl.kernel,
        mesh=mesh,
        compiler_params=pltpu.CompilerParams(use_tc_tiling_on_sc=False),
        out_type=jax.ShapeDtypeStruct((n_total, d), jnp.float32),
        scratch_types=[
            pltpu.VMEM((idx_rows_w, _ROW), jnp.int32),
            pltpu.VMEM((nbuf, chunk, d), jnp.float32),
            pltpu.SemaphoreType.DMA((nbuf,)),
            pltpu.SemaphoreType.DMA((nbuf,)),
        ],
    )
    def k(ids_hbm, table_hbm, out_hbm, idx_v, rows_v, gsem, wsem):
        wid = lax.axis_index("s") * nc + lax.axis_index("c")
        base = wid * n_per_w

        pltpu.sync_copy(ids_hbm.at[pl.ds(wid * idx_rows_w, idx_rows_w)], idx_v)

        def start_gather(g, b):
            for s in range(nstream):
                r = g * nstream + s
                pltpu.async_copy(
                    table_hbm.at[idx_v.at[r]],
                    rows_v.at[b].at[pl.ds(s * _ROW, _ROW)], gsem.at[b])

        def wait_gather(b):
            for s in range(nstream):
                pltpu.make_async_copy(
                    table_hbm.at[idx_v.at[0]],
                    rows_v.at[b].at[pl.ds(0, _ROW)], gsem.at[b]).wait()

        def start_write(g, b):
            off = pl.multiple_of(base + g * chunk, chunk)
            pltpu.async_copy(rows_v.at[b], out_hbm.at[pl.ds(off, chunk)],
                             wsem.at[b])

        def wait_write(b):
            pltpu.make_async_copy(rows_v.at[b],
                                  out_hbm.at[pl.ds(0, chunk)],
                                  wsem.at[b]).wait()

        for b in range(nbuf):
            start_gather(b, b)

        def body(i, carry):
            for b in range(nbuf):
                g = i * nbuf + b
                wait_gather(b)
                start_write(g, b)
                wait_write(b)
                start_gather(g + nbuf, b)
            return carry

        lax.fori_loop(0, outer - 1, body, 0)

        for b in range(nbuf):
            g = (outer - 1) * nbuf + b
            wait_gather(b)
            start_write(g, b)
        for b in range(nbuf):
            wait_write(b)

    return k


def kernel(input_ids, table):
    b, s = input_ids.shape
    d = table.shape[1]
    n = b * s
    ids2d = input_ids.reshape(n // _ROW, _ROW).astype(jnp.int32)
    out = _build_gather(n, d)(ids2d, table)
    return out.reshape(b, s, d)
